# baseline (speedup 1.0000x reference)
import functools

import jax
import jax.numpy as jnp
from jax import lax
from jax.experimental import pallas as pl
from jax.experimental.pallas import tpu as pltpu
from jax.experimental.pallas import tpu_sc as plsc

NC = 2
NS = 16
NW = NC * NS

CHUNK = 256
G = 1
R = G * CHUNK
W = 4


def _make_gather(total, d, chunks_per_w):
    per_w = total // NW
    nsc = per_w // R

    @functools.partial(
        pl.kernel,
        out_type=jax.ShapeDtypeStruct((total, d), jnp.float32),
        mesh=plsc.VectorSubcoreMesh(core_axis_name="c", subcore_axis_name="s"),
        scratch_types=(
            [pltpu.VMEM((chunks_per_w, CHUNK), jnp.int32),
             pltpu.VMEM((W * R, d), jnp.float32)]
            + [pltpu.SemaphoreType.DMA] * (2 * W)
        ),
        compiler_params=pltpu.CompilerParams(use_tc_tiling_on_sc=False),
    )
    def body(table_hbm, idx_hbm, out_hbm, idx_v, rows_v, *sems):
        wid = lax.axis_index("s") * NC + lax.axis_index("c")
        pltpu.sync_copy(idx_hbm.at[wid], idx_v)
        base = wid * per_w
        sems_g = sems[:W]
        sems_w = sems[W:]

        def fire_gather(sc, b):
            row = idx_v.at[sc]
            for g in range(R // 16):
                vidx = row[pl.ds(g * 16, 16)]
                pltpu.async_copy(
                    table_hbm.at[vidx],
                    rows_v.at[pl.ds(b * R + g * 16, 16)],
                    sems_g[b],
                )

        def wait_gather(b):
            pltpu.make_async_copy(
                table_hbm.at[idx_v.at[0]],
                rows_v.at[pl.ds(b * R, R)],
                sems_g[b],
            ).wait()

        def fire_write(sc, b):
            pltpu.async_copy(
                rows_v.at[pl.ds(b * R, R)],
                out_hbm.at[pl.ds(base + sc * R, R)],
                sems_w[b],
            )

        def wait_write(b):
            pltpu.make_async_copy(
                rows_v.at[pl.ds(b * R, R)],
                out_hbm.at[pl.ds(base, R)],
                sems_w[b],
            ).wait()

        for b in range(W):
            fire_gather(b, b)

        @pl.loop(0, nsc - W, step=W)
        def _main(j):
            for b in range(W):
                sc = j + b
                wait_gather(b)
                fire_write(sc, b)
                wait_write(b)
                fire_gather(sc + W, b)

        for b in range(W):
            wait_gather(b)
            fire_write(nsc - W + b, b)
        for b in range(W):
            wait_write(b)

    return body


def kernel(seq, embedding_weight):
    b, s = seq.shape
    _, d = embedding_weight.shape
    total = b * s
    per_w = total // NW
    chunks_per_w = per_w // CHUNK
    idx = seq.astype(jnp.int32).reshape(NW, chunks_per_w, CHUNK)
    out = _make_gather(total, d, chunks_per_w)(embedding_weight, idx)
    return out.reshape(b, s, d)


# speedup vs baseline: 1.0033x; 1.0033x over previous
import functools

import jax
import jax.numpy as jnp
from jax import lax
from jax.experimental import pallas as pl
from jax.experimental.pallas import tpu as pltpu
from jax.experimental.pallas import tpu_sc as plsc

NC = 2
NS = 16
NW = NC * NS

W = 4


def _make_gather(b, s, d):
    rows_per_w = b // NW

    @functools.partial(
        pl.kernel,
        out_type=jax.ShapeDtypeStruct((b, s, d), jnp.float32),
        mesh=plsc.VectorSubcoreMesh(core_axis_name="c", subcore_axis_name="s"),
        scratch_types=(
            [pltpu.VMEM((rows_per_w, s), jnp.int32),
             pltpu.VMEM((W, s, d), jnp.float32)]
            + [pltpu.SemaphoreType.DMA] * (2 * W)
        ),
        compiler_params=pltpu.CompilerParams(use_tc_tiling_on_sc=False),
    )
    def body(table_hbm, idx_hbm, out_hbm, idx_v, rows_v, *sems):
        wid = lax.axis_index("s") * NC + lax.axis_index("c")
        base = wid * rows_per_w
        pltpu.sync_copy(idx_hbm.at[pl.ds(base, rows_per_w)], idx_v)
        sems_g = sems[:W]
        sems_w = sems[W:]

        def fire_gather(r, slot):
            pltpu.async_copy(
                table_hbm.at[idx_v.at[r]], rows_v.at[slot], sems_g[slot]
            )

        def wait_gather(slot):
            pltpu.make_async_copy(
                table_hbm.at[idx_v.at[0]], rows_v.at[slot], sems_g[slot]
            ).wait()

        def fire_write(r, slot):
            pltpu.async_copy(
                rows_v.at[slot], out_hbm.at[base + r], sems_w[slot]
            )

        def wait_write(slot):
            pltpu.make_async_copy(
                rows_v.at[slot], out_hbm.at[base], sems_w[slot]
            ).wait()

        for slot in range(W):
            fire_gather(slot, slot)

        @pl.loop(0, rows_per_w - W, step=W)
        def _main(j):
            for slot in range(W):
                r = j + slot
                wait_gather(slot)
                fire_write(r, slot)
                wait_write(slot)
                fire_gather(r + W, slot)

        for slot in range(W):
            wait_gather(slot)
            fire_write(rows_per_w - W + slot, slot)
        for slot in range(W):
            wait_write(slot)

    return body


def kernel(seq, embedding_weight):
    b, s = seq.shape
    _, d = embedding_weight.shape
    return _make_gather(b, s, d)(embedding_weight, seq.astype(jnp.int32))
